# baseline (speedup 1.0000x reference)
import dataclasses
import functools

import jax
import jax.numpy as jnp
import numpy as np
from jax import lax
from jax.experimental import pallas as pl
from jax.experimental.pallas import tpu as pltpu
from jax.experimental.pallas import tpu_sc as plsc

_SIGMA_RATIO = 4.0
_SELU_SCALE = 1.0507009873554805
_SELU_ALPHA = 1.6732632423543772

_LANES = 16
_NW = 32
_NB = 8
_NBUF = 4


def _edge_weight(nd_flat, h_heads):
    ne = nd_flat.shape[0]

    def body(nd_ref, u_ref):
        nd = nd_ref[...]
        dmax = jnp.max(nd)
        c = (_SIGMA_RATIO * _SIGMA_RATIO) / (jnp.float32(h_heads) * dmax * dmax)
        u_ref[...] = jnp.exp(-(nd * nd) * c)

    return pl.pallas_call(
        body,
        out_shape=jax.ShapeDtypeStruct((ne,), jnp.float32),
    )(nd_flat)


def _sc_aggregate(xtw, nn_flat, u_flat, n, nt, k_nbr, f_row, h_heads):
    nseg = f_row // _LANES
    npair = f_row // (2 * _LANES)
    ngrp = (h_heads * f_row) // 128
    nchunks = nt // _NB
    mesh = plsc.VectorSubcoreMesh(core_axis_name="c", subcore_axis_name="s")
    cp = pltpu.CompilerParams()
    for fld, val in (("needs_layout_passes", False),
                     ("use_tc_tiling_on_sc", False)):
        if fld in pltpu.CompilerParams.__dataclass_fields__:
            cp = dataclasses.replace(cp, **{fld: val})

    ec = _NB * k_nbr
    nw = f_row // 2

    scratch = []
    for _ in range(_NBUF):
        scratch += [pltpu.VMEM((ec,), jnp.int32),
                    pltpu.VMEM((ec,), jnp.float32),
                    pltpu.VMEM((ec, nw), jnp.int32),
                    pltpu.VMEM((ngrp, _NB, 128), jnp.float32),
                    pltpu.SemaphoreType.DMA,
                    pltpu.SemaphoreType.DMA]

    @functools.partial(
        pl.kernel,
        mesh=mesh,
        compiler_params=cp,
        out_type=jax.ShapeDtypeStruct((n // _NB, ngrp, _NB, 128), jnp.float32),
        scratch_types=scratch,
    )
    def sc_kernel(xt_hbm, nn_hbm, u_hbm, agg_hbm, *bufs_flat):
        wid = lax.axis_index("s") * 2 + lax.axis_index("c")
        node_base = wid * nt
        bufs = tuple(bufs_flat[6 * i:6 * i + 6] for i in range(_NBUF))

        def clamped(g):
            return jnp.minimum(node_base + g * _NB, n - _NB)

        def start_gather(g, buf):
            idx_v, u_v, rows_v, _, sem, _ = buf
            e0 = clamped(g) * k_nbr
            pltpu.sync_copy(nn_hbm.at[pl.ds(e0, ec)], idx_v)
            pltpu.sync_copy(u_hbm.at[pl.ds(e0, ec)], u_v)
            pltpu.async_copy(xt_hbm.at[idx_v], rows_v, sem)

        def do_chunk(g, p, buf):
            idx_v, u_v, rows_v, out_v, sem, osem = buf
            cidx = clamped(g) // _NB
            pltpu.make_async_copy(xt_hbm.at[idx_v], rows_v, sem).wait()

            @pl.when(p > 0)
            def _():
                pltpu.make_async_copy(out_v, agg_hbm.at[cidx], osem).wait()

            @pl.loop(0, _NB)
            def _node(i):
                acc = [jnp.zeros((_LANES,), jnp.float32)
                       for _ in range(h_heads * nseg)]
                for k in range(k_nbr):
                    e = i * k_nbr + k
                    ub = plsc.load_gather(
                        u_v, [jnp.full((_LANES,), e, jnp.int32)])
                    for b3 in range(npair):
                        wv = rows_v[e, pl.ds(b3 * _LANES, _LANES)]
                        evens = plsc.bitcast(
                            jnp.left_shift(wv, 16), jnp.float32)
                        odds = plsc.bitcast(
                            jnp.bitwise_and(wv, jnp.int32(-65536)),
                            jnp.float32)
                        for half, seg in ((0, evens), (1, odds)):
                            s = 2 * b3 + half
                            pw = seg
                            for h in range(h_heads):
                                pw = ub * pw
                                acc[h * nseg + s] = acc[h * nseg + s] + pw
                for h in range(h_heads):
                    for s in range(nseg):
                        pos = h * f_row + s * _LANES
                        out_v[pos // 128, i, pl.ds(pos % 128, _LANES)] = (
                            acc[h * nseg + s])

            pltpu.async_copy(out_v, agg_hbm.at[cidx], osem)

        for b in range(_NBUF):
            start_gather(jnp.int32(b), bufs[b])

        @pl.loop(0, nchunks // _NBUF)
        def _group(p):
            g0 = p * _NBUF
            for b in range(_NBUF):
                g = g0 + b
                do_chunk(g, p, bufs[b])

                @pl.when(g + _NBUF < nchunks)
                def _():
                    start_gather(g + _NBUF, bufs[b])

        for b in range(_NBUF):
            pltpu.make_async_copy(
                bufs[b][3],
                agg_hbm.at[clamped(nchunks - _NBUF + b) // _NB],
                bufs[b][5]).wait()

    return sc_kernel(xtw, nn_flat, u_flat)


def _shrink_selu(agg4, wbig, bbig, n):
    nchunk, ngrp = agg4.shape[0], agg4.shape[1]
    oc = wbig.shape[1]
    cblk = 250
    blk = cblk * _NB

    def body(agg_ref, w_ref, b_ref, y_ref):
        y = jnp.zeros((blk, oc), jnp.float32) + b_ref[...]
        for g in range(ngrp):
            ag = agg_ref[:, g, :, :].reshape(blk, 128)
            y = y + jnp.dot(ag, w_ref[pl.ds(g * 128, 128), :],
                            preferred_element_type=jnp.float32)
        y_ref[...] = _SELU_SCALE * jnp.where(
            y > 0, y, _SELU_ALPHA * (jnp.exp(y) - 1.0))

    return pl.pallas_call(
        body,
        grid=(nchunk // cblk,),
        in_specs=[
            pl.BlockSpec((cblk, ngrp, _NB, 128), lambda i: (i, 0, 0, 0)),
            pl.BlockSpec((ngrp * 128, oc), lambda i: (0, 0)),
            pl.BlockSpec((1, oc), lambda i: (0, 0)),
        ],
        out_specs=pl.BlockSpec((blk, oc), lambda i: (i, 0)),
        out_shape=jax.ShapeDtypeStruct((n, oc), jnp.float32),
    )(agg4, wbig, bbig)


def kernel(x, nearest_dists, shrink_w, shrink_b, nearest_nodes):
    b, t_in, n, c = x.shape
    _, k_nbr = nearest_nodes.shape
    t_out = shrink_w.shape[0]
    h_heads = shrink_w.shape[1] // t_in
    f_row = t_in * c
    nseg = f_row // _LANES
    nt = -(-n // (_NW * _NBUF * _NB)) * _NBUF * _NB

    nn_flat = nearest_nodes.reshape(-1)
    u_flat = _edge_weight(nearest_dists.reshape(-1), h_heads)
    xt16 = x[0].transpose(1, 0, 2).reshape(n, f_row).astype(jnp.bfloat16)
    xtw = jax.lax.bitcast_convert_type(
        xt16.reshape(n, f_row // 2, 2), jnp.int32)

    agg4 = _sc_aggregate(xtw, nn_flat, u_flat, n, nt, k_nbr, f_row, h_heads)

    a = shrink_w.reshape(t_out, t_in, h_heads)
    wbig = jnp.einsum('oth,cd->htcod', a,
                      jnp.eye(c, dtype=jnp.float32)).reshape(
                          h_heads * f_row, t_out * c)
    perm = np.empty((h_heads, nseg, _LANES), np.int32)
    for h in range(h_heads):
        for s in range(nseg):
            for l in range(_LANES):
                perm[h, s, l] = h * f_row + 32 * (s // 2) + 2 * l + (s % 2)
    wbig = wbig[perm.reshape(-1)]
    bbig = jnp.repeat(shrink_b, c).reshape(1, t_out * c)

    y = _shrink_selu(agg4, wbig, bbig, n)

    y = y.reshape(n, t_out, c).transpose(1, 0, 2)[None]
    return jnp.concatenate([x, y], axis=1)


# speedup vs baseline: 1.4086x; 1.4086x over previous
import dataclasses
import functools

import jax
import jax.numpy as jnp
import numpy as np
from jax import lax
from jax.experimental import pallas as pl
from jax.experimental.pallas import tpu as pltpu
from jax.experimental.pallas import tpu_sc as plsc

_SIGMA_RATIO = 4.0
_SELU_SCALE = 1.0507009873554805
_SELU_ALPHA = 1.6732632423543772

_LANES = 16
_NW = 32
_NB = 8
_NBUF = 4


def _edge_weight(nd_flat, h_heads):
    ne = nd_flat.shape[0]

    def body(nd_ref, u_ref):
        nd = nd_ref[...]
        dmax = jnp.max(nd)
        c = (_SIGMA_RATIO * _SIGMA_RATIO) / (jnp.float32(h_heads) * dmax * dmax)
        u_ref[...] = jnp.exp(-(nd * nd) * c)

    return pl.pallas_call(
        body,
        out_shape=jax.ShapeDtypeStruct((ne,), jnp.float32),
    )(nd_flat)


def _sc_aggregate(xt16, nn_flat, u_flat, n, nt, k_nbr, f_row, h_heads):
    nseg = f_row // _LANES
    npair = f_row // (2 * _LANES)
    ngrp = (h_heads * f_row) // 128
    nchunks = nt // _NB
    mesh = plsc.VectorSubcoreMesh(core_axis_name="c", subcore_axis_name="s")
    cp = pltpu.CompilerParams()
    for fld, val in (("needs_layout_passes", False),
                     ("use_tc_tiling_on_sc", False)):
        if fld in pltpu.CompilerParams.__dataclass_fields__:
            cp = dataclasses.replace(cp, **{fld: val})

    ec = _NB * k_nbr

    scratch = []
    for _ in range(_NBUF):
        scratch += [pltpu.VMEM((ec,), jnp.int32),
                    pltpu.VMEM((ec,), jnp.float32),
                    pltpu.VMEM((ec, f_row), jnp.bfloat16),
                    pltpu.VMEM((ngrp, _NB, 128), jnp.float32),
                    pltpu.SemaphoreType.DMA,
                    pltpu.SemaphoreType.DMA,
                    pltpu.SemaphoreType.DMA]

    @functools.partial(
        pl.kernel,
        mesh=mesh,
        compiler_params=cp,
        out_type=jax.ShapeDtypeStruct((n // _NB, ngrp, _NB, 128), jnp.float32),
        scratch_types=scratch,
    )
    def sc_kernel(xt_hbm, nn_hbm, u_hbm, agg_hbm, *bufs_flat):
        wid = lax.axis_index("s") * 2 + lax.axis_index("c")
        node_base = wid * nt
        bufs = tuple(bufs_flat[7 * i:7 * i + 7] for i in range(_NBUF))

        def clamped(g):
            return jnp.minimum(node_base + g * _NB, n - _NB)

        def load_inputs(g, buf):
            idx_v, u_v, sem_in = buf[0], buf[1], buf[4]
            e0 = clamped(g) * k_nbr
            pltpu.async_copy(nn_hbm.at[pl.ds(e0, ec)], idx_v, sem_in)
            pltpu.async_copy(u_hbm.at[pl.ds(e0, ec)], u_v, sem_in)

        def wait_inputs(g, buf):
            idx_v, u_v, sem_in = buf[0], buf[1], buf[4]
            e0 = clamped(g) * k_nbr
            pltpu.make_async_copy(
                nn_hbm.at[pl.ds(e0, ec)], idx_v, sem_in).wait()
            pltpu.make_async_copy(
                u_hbm.at[pl.ds(e0, ec)], u_v, sem_in).wait()

        def start_gather(buf):
            pltpu.async_copy(xt_hbm.at[buf[0]], buf[2], buf[5])

        def do_chunk(g, p, buf):
            idx_v, u_v, rows_v, out_v, _, sem_g, osem = buf
            cidx = clamped(g) // _NB
            pltpu.make_async_copy(xt_hbm.at[idx_v], rows_v, sem_g).wait()

            @pl.when(p > 0)
            def _():
                pltpu.make_async_copy(out_v, agg_hbm.at[cidx], osem).wait()

            @pl.loop(0, _NB)
            def _node(i):
                acc = [jnp.zeros((_LANES,), jnp.float32)
                       for _ in range(h_heads * nseg)]
                for k in range(k_nbr):
                    e = i * k_nbr + k
                    ub = plsc.load_gather(
                        u_v, [jnp.full((_LANES,), e, jnp.int32)])
                    for b3 in range(npair):
                        blk = rows_v[e, pl.ds(b3 * 2 * _LANES, 2 * _LANES)]
                        fa, fb = plsc.unpack(
                            blk, format=plsc.PackFormat.INTERLEAVED,
                            preferred_element_type=jnp.float32)
                        for half, seg in ((0, fa), (1, fb)):
                            s = 2 * b3 + half
                            pw = seg
                            for h in range(h_heads):
                                pw = ub * pw
                                acc[h * nseg + s] = acc[h * nseg + s] + pw
                for h in range(h_heads):
                    for s in range(nseg):
                        pos = h * f_row + s * _LANES
                        out_v[pos // 128, i, pl.ds(pos % 128, _LANES)] = (
                            acc[h * nseg + s])

            pltpu.async_copy(out_v, agg_hbm.at[cidx], osem)

        for b in range(_NBUF):
            load_inputs(jnp.int32(b), bufs[b])
        for b in range(2):
            wait_inputs(jnp.int32(b), bufs[b])
            start_gather(bufs[b])

        @pl.loop(0, nchunks // _NBUF)
        def _group(p):
            g0 = p * _NBUF
            for b in range(_NBUF):
                g = g0 + b
                buf = bufs[b]
                do_chunk(g, p, buf)

                @pl.when(g + _NBUF < nchunks)
                def _():
                    load_inputs(g + _NBUF, buf)

                @pl.when(g + 2 < nchunks)
                def _():
                    b2 = bufs[(b + 2) % _NBUF]
                    wait_inputs(g + 2, b2)
                    start_gather(b2)

        for b in range(_NBUF):
            pltpu.make_async_copy(
                bufs[b][3],
                agg_hbm.at[clamped(nchunks - _NBUF + b) // _NB],
                bufs[b][6]).wait()

    return sc_kernel(xt16, nn_flat, u_flat)


def _shrink_selu(agg4, wbig, bbig, n):
    nchunk, ngrp = agg4.shape[0], agg4.shape[1]
    oc = wbig.shape[1]
    cblk = 250
    blk = cblk * _NB

    def body(agg_ref, w_ref, b_ref, y_ref):
        y = jnp.zeros((blk, oc), jnp.float32) + b_ref[...]
        for g in range(ngrp):
            ag = agg_ref[:, g, :, :].reshape(blk, 128)
            y = y + jnp.dot(ag, w_ref[pl.ds(g * 128, 128), :],
                            preferred_element_type=jnp.float32)
        y_ref[...] = _SELU_SCALE * jnp.where(
            y > 0, y, _SELU_ALPHA * (jnp.exp(y) - 1.0))

    return pl.pallas_call(
        body,
        grid=(nchunk // cblk,),
        in_specs=[
            pl.BlockSpec((cblk, ngrp, _NB, 128), lambda i: (i, 0, 0, 0)),
            pl.BlockSpec((ngrp * 128, oc), lambda i: (0, 0)),
            pl.BlockSpec((1, oc), lambda i: (0, 0)),
        ],
        out_specs=pl.BlockSpec((blk, oc), lambda i: (i, 0)),
        out_shape=jax.ShapeDtypeStruct((n, oc), jnp.float32),
    )(agg4, wbig, bbig)


def kernel(x, nearest_dists, shrink_w, shrink_b, nearest_nodes):
    b, t_in, n, c = x.shape
    _, k_nbr = nearest_nodes.shape
    t_out = shrink_w.shape[0]
    h_heads = shrink_w.shape[1] // t_in
    f_row = t_in * c
    nseg = f_row // _LANES
    nt = -(-n // (_NW * _NBUF * _NB)) * _NBUF * _NB

    nn_flat = nearest_nodes.reshape(-1)
    u_flat = _edge_weight(nearest_dists.reshape(-1), h_heads)
    xt16 = x[0].transpose(1, 0, 2).reshape(n, f_row).astype(jnp.bfloat16)

    agg4 = _sc_aggregate(xt16, nn_flat, u_flat, n, nt, k_nbr, f_row, h_heads)

    a = shrink_w.reshape(t_out, t_in, h_heads)
    wbig = jnp.einsum('oth,cd->htcod', a,
                      jnp.eye(c, dtype=jnp.float32)).reshape(
                          h_heads * f_row, t_out * c)
    perm = np.empty((h_heads, nseg, _LANES), np.int32)
    for h in range(h_heads):
        for s in range(nseg):
            for l in range(_LANES):
                perm[h, s, l] = h * f_row + 32 * (s // 2) + 2 * l + (s % 2)
    wbig = wbig[perm.reshape(-1)]
    bbig = jnp.repeat(shrink_b, c).reshape(1, t_out * c)

    y = _shrink_selu(agg4, wbig, bbig, n)

    y = y.reshape(n, t_out, c).transpose(1, 0, 2)[None]
    return jnp.concatenate([x, y], axis=1)
